# baseline (speedup 1.0000x reference)
import functools

import jax
import jax.numpy as jnp
from jax import lax
from jax.experimental import pallas as pl
from jax.experimental.pallas import tpu as pltpu
from jax.experimental.pallas import tpu_sc as plsc

_REDUCE = 4
_K = 32
_RADI = 0.2


def _fps_body(planes_ref, out_ref):
    B = planes_ref.shape[0]
    G = out_ref.shape[1]
    sub = lax.broadcasted_iota(jnp.int32, (64, 128), 0)
    lane = lax.broadcasted_iota(jnp.int32, (64, 128), 1)
    iota = sub * 128 + lane
    xs = [planes_ref[b, 0] for b in range(B)]
    ys = [planes_ref[b, 1] for b in range(B)]
    zs = [planes_ref[b, 2] for b in range(B)]

    def body(i, dists):
        new_d = []
        for b in range(B):
            db = dists[b]
            m = jnp.max(db)
            idxb = jnp.min(jnp.where(db == m, iota, jnp.int32(1 << 30)))
            sel = iota == idxb
            cx = jnp.sum(jnp.where(sel, xs[b], 0.0))
            cy = jnp.sum(jnp.where(sel, ys[b], 0.0))
            cz = jnp.sum(jnp.where(sel, zs[b], 0.0))
            row = jnp.concatenate(
                [cx.reshape(1, 1), cy.reshape(1, 1), cz.reshape(1, 1)], axis=1
            )
            out_ref[b, pl.ds(i, 1), :] = row
            dx = xs[b] - cx
            dy = ys[b] - cy
            dz = zs[b] - cz
            d = (dx * dx + dy * dy) + dz * dz
            new_d.append(jnp.minimum(db, d))
        return jnp.stack(new_d)

    dists0 = jnp.full((B, 64, 128), 1e10, dtype=jnp.float32)
    lax.fori_loop(0, G, body, dists0)


def _fps_new_xyz(xyz, G):
    B, N, _ = xyz.shape
    planes = xyz.transpose(0, 2, 1).reshape(B, 3, N // 128, 128)
    return pl.pallas_call(
        _fps_body,
        out_shape=jax.ShapeDtypeStruct((B, G, 3), jnp.float32),
    )(planes)


def _make_group_sc(B, N, G, D):
    NW = 32
    CPB = NW // B
    QPT = G // CPB
    L = 16
    JMAX = N // L
    RSQ = jnp.float32(_RADI * _RADI)
    mesh = plsc.VectorSubcoreMesh(core_axis_name="c", subcore_axis_name="s")

    @functools.partial(
        pl.kernel,
        mesh=mesh,
        compiler_params=pltpu.CompilerParams(needs_layout_passes=False),
        out_type=jax.ShapeDtypeStruct((B * G, D), jnp.float32),
        scratch_types=[
            pltpu.VMEM((N,), jnp.float32),
            pltpu.VMEM((N,), jnp.float32),
            pltpu.VMEM((N,), jnp.float32),
            pltpu.VMEM((QPT,), jnp.float32),
            pltpu.VMEM((QPT,), jnp.float32),
            pltpu.VMEM((QPT,), jnp.float32),
            pltpu.VMEM((QPT, _K), jnp.int32),
            pltpu.VMEM((D,), jnp.float32),
            pltpu.VMEM((D,), jnp.float32),
            pltpu.VMEM((2, _K, D), jnp.float32),
            pltpu.VMEM((QPT, D), jnp.float32),
            pltpu.SemaphoreType.DMA,
            pltpu.SemaphoreType.DMA,
        ],
    )
    def grouper(xyz_hbm, q_hbm, pts_hbm, al_hbm, be_hbm, out_hbm,
                x_v, y_v, z_v, qx_v, qy_v, qz_v, idx_v, al_v, be_v,
                rows_v, res_v, sem0, sem1):
        wid = lax.axis_index("s") * 2 + lax.axis_index("c")
        b = wid // CPB
        qbase = (wid % CPB) * QPT
        rowbase = b * N

        pltpu.sync_copy(xyz_hbm.at[pl.ds((b * 3 + 0) * N, N)], x_v)
        pltpu.sync_copy(xyz_hbm.at[pl.ds((b * 3 + 1) * N, N)], y_v)
        pltpu.sync_copy(xyz_hbm.at[pl.ds((b * 3 + 2) * N, N)], z_v)
        pltpu.sync_copy(q_hbm.at[pl.ds((b * 3 + 0) * G + qbase, QPT)], qx_v)
        pltpu.sync_copy(q_hbm.at[pl.ds((b * 3 + 1) * G + qbase, QPT)], qy_v)
        pltpu.sync_copy(q_hbm.at[pl.ds((b * 3 + 2) * G + qbase, QPT)], qz_v)
        pltpu.sync_copy(al_hbm, al_v)
        pltpu.sync_copy(be_hbm, be_v)

        lanes = lax.iota(jnp.int32, L)

        def bq_block(qb, carry):
            qxv = qx_v[pl.ds(qb * L, L)]
            qyv = qy_v[pl.ds(qb * L, L)]
            qzv = qz_v[pl.ds(qb * L, L)]
            for qi in range(L):
                q = qb * L + qi
                qx = qxv[qi]
                qy = qyv[qi]
                qz = qzv[qi]

                def body(j, cnt):
                    px = x_v[pl.ds(j * L, L)]
                    py = y_v[pl.ds(j * L, L)]
                    pz = z_v[pl.ds(j * L, L)]
                    dx = qx - px
                    dy = qy - py
                    dz = qz - pz
                    d2 = (dx * dx + dy * dy) + dz * dz
                    m = d2 < RSQ
                    mi = jnp.where(m, jnp.int32(1), jnp.int32(0))
                    nh = jnp.sum(mi)
                    incl = plsc.cumsum(mi)
                    pos = cnt + incl - 1
                    wm = m & (pos < _K)
                    inds = rowbase + j * L + lanes
                    plsc.store_scatter(
                        idx_v, [jnp.full((L,), q, jnp.int32), pos], inds,
                        mask=wm,
                    )
                    return cnt + nh

                cnt = lax.fori_loop(0, JMAX, body, jnp.int32(0))

                @pl.when(cnt < _K)
                def _():
                    first = jnp.full((L,), idx_v[q, pl.ds(0, L)][0], jnp.int32)
                    for half in range(_K // L):
                        sl = lanes + half * L
                        plsc.store_scatter(
                            idx_v, [jnp.full((L,), q, jnp.int32), sl], first,
                            mask=sl >= cnt,
                        )

            return carry

        lax.fori_loop(0, QPT // L, bq_block, jnp.int32(0))

        al_blk = [al_v[pl.ds(k * L, L)] for k in range(D // L)]
        be_blk = [be_v[pl.ds(k * L, L)] for k in range(D // L)]
        sems = [sem0, sem1]

        def gather(g, s):
            return pltpu.make_async_copy(
                pts_hbm.at[idx_v.at[g]], rows_v.at[s], sems[s]
            )

        gather(jnp.int32(0), 0).start()
        gather(jnp.int32(1), 1).start()

        def grp_pair(gg, carry):
            for s in range(2):
                g = gg * 2 + s
                gather(g, s).wait()
                for k in range(D // L):
                    asum = jnp.zeros((L,), jnp.float32)
                    amax = jnp.full((L,), -jnp.inf, jnp.float32)
                    for r in range(_K):
                        v = rows_v[s, r, pl.ds(k * L, L)]
                        asum = asum + v
                        amax = jnp.maximum(amax, al_blk[k] * v)
                    mean = asum * jnp.float32(1.0 / _K)
                    res_v[g, pl.ds(k * L, L)] = (
                        amax + (be_blk[k] - al_blk[k] * mean)
                    )

                @pl.when(g + 2 < QPT)
                def _():
                    gather(g + 2, s).start()
            return carry

        lax.fori_loop(0, QPT // 2, grp_pair, jnp.int32(0))

        pltpu.sync_copy(res_v, out_hbm.at[pl.ds(b * G + qbase, QPT)])

    return grouper


def kernel(xyz, points, affine_alpha, affine_beta):
    B, N, D = points.shape
    G = N // _REDUCE
    new_xyz = _fps_new_xyz(xyz, G)

    xyz_t = xyz.transpose(0, 2, 1)
    newxyz_t = new_xyz.transpose(0, 2, 1)
    feats = _make_group_sc(B, N, G, D)(
        xyz_t.reshape(-1),
        newxyz_t.reshape(-1),
        points.reshape(B * N, D),
        affine_alpha.reshape(D),
        affine_beta.reshape(D),
    )
    feats = feats.reshape(B, G, D)
    new_points = jnp.concatenate([feats.transpose(0, 2, 1), newxyz_t], axis=1)
    return (new_xyz, new_points)


# speedup vs baseline: 1.4572x; 1.4572x over previous
import functools

import jax
import jax.numpy as jnp
from jax import lax
from jax.experimental import pallas as pl
from jax.experimental.pallas import tpu as pltpu
from jax.experimental.pallas import tpu_sc as plsc

_REDUCE = 4
_K = 32
_RADI = 0.2


def _fps_body(planes_ref, out_ref):
    B = planes_ref.shape[0]
    G = out_ref.shape[1]
    sub = lax.broadcasted_iota(jnp.int32, (64, 128), 0)
    lane = lax.broadcasted_iota(jnp.int32, (64, 128), 1)
    iota = sub * 128 + lane
    xs = [planes_ref[b, 0] for b in range(B)]
    ys = [planes_ref[b, 1] for b in range(B)]
    zs = [planes_ref[b, 2] for b in range(B)]

    def body(i, dists):
        new_d = []
        for b in range(B):
            db = dists[b]
            m = jnp.max(db)
            idxb = jnp.min(jnp.where(db == m, iota, jnp.int32(1 << 30)))
            sel = iota == idxb
            cx = jnp.sum(jnp.where(sel, xs[b], 0.0))
            cy = jnp.sum(jnp.where(sel, ys[b], 0.0))
            cz = jnp.sum(jnp.where(sel, zs[b], 0.0))
            row = jnp.concatenate(
                [cx.reshape(1, 1), cy.reshape(1, 1), cz.reshape(1, 1)], axis=1
            )
            out_ref[b, pl.ds(i, 1), :] = row
            dx = xs[b] - cx
            dy = ys[b] - cy
            dz = zs[b] - cz
            d = (dx * dx + dy * dy) + dz * dz
            new_d.append(jnp.minimum(db, d))
        return jnp.stack(new_d)

    dists0 = jnp.full((B, 64, 128), 1e10, dtype=jnp.float32)
    lax.fori_loop(0, G, body, dists0)


def _fps_new_xyz(xyz, G):
    B, N, _ = xyz.shape
    planes = xyz.transpose(0, 2, 1).reshape(B, 3, N // 128, 128)
    return pl.pallas_call(
        _fps_body,
        out_shape=jax.ShapeDtypeStruct((B, G, 3), jnp.float32),
    )(planes)


def _make_group_sc(B, N, G, D):
    NW = 32
    CPB = NW // B
    QPT = G // CPB
    L = 16
    JMAX = N // L
    RSQ = jnp.float32(_RADI * _RADI)
    mesh = plsc.VectorSubcoreMesh(core_axis_name="c", subcore_axis_name="s")

    @functools.partial(
        pl.kernel,
        mesh=mesh,
        compiler_params=pltpu.CompilerParams(needs_layout_passes=False),
        out_type=jax.ShapeDtypeStruct((B * G, D), jnp.float32),
        scratch_types=[
            pltpu.VMEM((N,), jnp.float32),
            pltpu.VMEM((N,), jnp.float32),
            pltpu.VMEM((N,), jnp.float32),
            pltpu.VMEM((QPT,), jnp.float32),
            pltpu.VMEM((QPT,), jnp.float32),
            pltpu.VMEM((QPT,), jnp.float32),
            pltpu.VMEM((QPT, _K), jnp.int32),
            pltpu.VMEM((D,), jnp.float32),
            pltpu.VMEM((D,), jnp.float32),
            pltpu.VMEM((2, _K, D), jnp.float32),
            pltpu.VMEM((QPT, D), jnp.float32),
            pltpu.SMEM((1,), jnp.int32),
            pltpu.SemaphoreType.DMA,
            pltpu.SemaphoreType.DMA,
        ],
    )
    def grouper(xyz_hbm, q_hbm, pts_hbm, al_hbm, be_hbm, out_hbm,
                x_v, y_v, z_v, qx_v, qy_v, qz_v, idx_v, al_v, be_v,
                rows_v, res_v, cnt_ref, sem0, sem1):
        wid = lax.axis_index("s") * 2 + lax.axis_index("c")
        b = wid // CPB
        qbase = (wid % CPB) * QPT
        rowbase = b * N

        pltpu.sync_copy(xyz_hbm.at[pl.ds((b * 3 + 0) * N, N)], x_v)
        pltpu.sync_copy(xyz_hbm.at[pl.ds((b * 3 + 1) * N, N)], y_v)
        pltpu.sync_copy(xyz_hbm.at[pl.ds((b * 3 + 2) * N, N)], z_v)
        pltpu.sync_copy(q_hbm.at[pl.ds((b * 3 + 0) * G + qbase, QPT)], qx_v)
        pltpu.sync_copy(q_hbm.at[pl.ds((b * 3 + 1) * G + qbase, QPT)], qy_v)
        pltpu.sync_copy(q_hbm.at[pl.ds((b * 3 + 2) * G + qbase, QPT)], qz_v)
        pltpu.sync_copy(al_hbm, al_v)
        pltpu.sync_copy(be_hbm, be_v)

        lanes = lax.iota(jnp.int32, L)

        CHUNK = 8

        def bq_block(qb, carry):
            qxv = qx_v[pl.ds(qb * L, L)]
            qyv = qy_v[pl.ds(qb * L, L)]
            qzv = qz_v[pl.ds(qb * L, L)]
            for qi in range(L):
                q = qb * L + qi
                qx = qxv[qi]
                qy = qyv[qi]
                qz = qzv[qi]
                cnt_ref[0] = jnp.int32(0)

                def chunk_body(c, carry2):
                    cnt0 = cnt_ref[0]

                    @pl.when(cnt0 < _K)
                    def _():
                        cnt = cnt0
                        for jj in range(CHUNK):
                            j = c * CHUNK + jj
                            px = x_v[pl.ds(j * L, L)]
                            py = y_v[pl.ds(j * L, L)]
                            pz = z_v[pl.ds(j * L, L)]
                            dx = qx - px
                            dy = qy - py
                            dz = qz - pz
                            d2 = (dx * dx + dy * dy) + dz * dz
                            m = d2 < RSQ
                            mi = jnp.where(m, jnp.int32(1), jnp.int32(0))
                            nh = jnp.sum(mi)
                            incl = plsc.cumsum(mi)
                            pos = cnt + incl - 1
                            wm = m & (pos < _K)
                            inds = rowbase + j * L + lanes
                            plsc.store_scatter(
                                idx_v, [jnp.full((L,), q, jnp.int32), pos],
                                inds, mask=wm,
                            )
                            cnt = cnt + nh
                        cnt_ref[0] = cnt

                    return carry2

                lax.fori_loop(0, JMAX // CHUNK, chunk_body, jnp.int32(0))
                cnt = cnt_ref[0]

                @pl.when(cnt < _K)
                def _():
                    first = jnp.full((L,), idx_v[q, pl.ds(0, L)][0], jnp.int32)
                    for half in range(_K // L):
                        sl = lanes + half * L
                        plsc.store_scatter(
                            idx_v, [jnp.full((L,), q, jnp.int32), sl], first,
                            mask=sl >= cnt,
                        )

            return carry

        lax.fori_loop(0, QPT // L, bq_block, jnp.int32(0))

        al_blk = [al_v[pl.ds(k * L, L)] for k in range(D // L)]
        be_blk = [be_v[pl.ds(k * L, L)] for k in range(D // L)]
        sems = [sem0, sem1]

        def gather(g, s):
            return pltpu.make_async_copy(
                pts_hbm.at[idx_v.at[g]], rows_v.at[s], sems[s]
            )

        gather(jnp.int32(0), 0).start()
        gather(jnp.int32(1), 1).start()

        def grp_pair(gg, carry):
            for s in range(2):
                g = gg * 2 + s
                gather(g, s).wait()
                for k in range(D // L):
                    asum = jnp.zeros((L,), jnp.float32)
                    amax = jnp.full((L,), -jnp.inf, jnp.float32)
                    for r in range(_K):
                        v = rows_v[s, r, pl.ds(k * L, L)]
                        asum = asum + v
                        amax = jnp.maximum(amax, al_blk[k] * v)
                    mean = asum * jnp.float32(1.0 / _K)
                    res_v[g, pl.ds(k * L, L)] = (
                        amax + (be_blk[k] - al_blk[k] * mean)
                    )

                @pl.when(g + 2 < QPT)
                def _():
                    gather(g + 2, s).start()
            return carry

        lax.fori_loop(0, QPT // 2, grp_pair, jnp.int32(0))

        pltpu.sync_copy(res_v, out_hbm.at[pl.ds(b * G + qbase, QPT)])

    return grouper


def kernel(xyz, points, affine_alpha, affine_beta):
    B, N, D = points.shape
    G = N // _REDUCE
    new_xyz = _fps_new_xyz(xyz, G)

    xyz_t = xyz.transpose(0, 2, 1)
    newxyz_t = new_xyz.transpose(0, 2, 1)
    feats = _make_group_sc(B, N, G, D)(
        xyz_t.reshape(-1),
        newxyz_t.reshape(-1),
        points.reshape(B * N, D),
        affine_alpha.reshape(D),
        affine_beta.reshape(D),
    )
    feats = feats.reshape(B, G, D)
    new_points = jnp.concatenate([feats.transpose(0, 2, 1), newxyz_t], axis=1)
    return (new_xyz, new_points)


# speedup vs baseline: 1.7317x; 1.1884x over previous
import functools

import jax
import jax.numpy as jnp
from jax import lax
from jax.experimental import pallas as pl
from jax.experimental.pallas import tpu as pltpu
from jax.experimental.pallas import tpu_sc as plsc

_REDUCE = 4
_K = 32
_RADI = 0.2


def _fps_body(planes_ref, out_ref):
    B = planes_ref.shape[0]
    G = out_ref.shape[1]
    sub = lax.broadcasted_iota(jnp.int32, (8, 128), 0)
    lane = lax.broadcasted_iota(jnp.int32, (8, 128), 1)
    vbase = sub * 128 + lane
    lane_row = lax.broadcasted_iota(jnp.int32, (1, 128), 1)

    def comb(a, c):
        va, ia = a
        vc, ic = c
        keep = (va > vc) | ((va == vc) & (ia < ic))
        return (jnp.where(keep, va, vc), jnp.where(keep, ia, ic))

    def body(i, dists):
        new_d = []
        for b in range(B):
            db = dists[b]
            vi = [(db[k * 8:(k + 1) * 8, :], vbase + k * 1024) for k in range(8)]
            while len(vi) > 1:
                vi = [comb(vi[j], vi[j + 1]) for j in range(0, len(vi), 2)]
            v8, i8 = vi[0]
            for sh in (4, 2, 1):
                v8, i8 = comb((v8, i8), (pltpu.roll(v8, sh, 0),
                                         pltpu.roll(i8, sh, 0)))
            for sh in (64, 32, 16, 8, 4, 2, 1):
                v8, i8 = comb((v8, i8), (pltpu.roll(v8, sh, 1),
                                         pltpu.roll(i8, sh, 1)))
            idx = i8[0, 0]
            r = idx // 128
            c = idx % 128
            lm = lane_row == c
            xrow = planes_ref[b, 0, pl.ds(r, 1), :]
            yrow = planes_ref[b, 1, pl.ds(r, 1), :]
            zrow = planes_ref[b, 2, pl.ds(r, 1), :]
            cx = jnp.sum(jnp.where(lm, xrow, 0.0))
            cy = jnp.sum(jnp.where(lm, yrow, 0.0))
            cz = jnp.sum(jnp.where(lm, zrow, 0.0))
            row = jnp.concatenate(
                [cx.reshape(1, 1), cy.reshape(1, 1), cz.reshape(1, 1)], axis=1
            )
            out_ref[b, pl.ds(i, 1), :] = row
            dx = planes_ref[b, 0] - cx
            dy = planes_ref[b, 1] - cy
            dz = planes_ref[b, 2] - cz
            d = (dx * dx + dy * dy) + dz * dz
            new_d.append(jnp.minimum(db, d))
        return tuple(new_d)

    dists0 = tuple(
        jnp.full((64, 128), 1e10, dtype=jnp.float32) for _ in range(B)
    )
    lax.fori_loop(0, G, body, dists0)


def _fps_new_xyz(xyz, G):
    B, N, _ = xyz.shape
    planes = xyz.transpose(0, 2, 1).reshape(B, 3, N // 128, 128)
    return pl.pallas_call(
        _fps_body,
        out_shape=jax.ShapeDtypeStruct((B, G, 3), jnp.float32),
    )(planes)


def _make_group_sc(B, N, G, D):
    NW = 32
    CPB = NW // B
    QPT = G // CPB
    L = 16
    JMAX = N // L
    RSQ = jnp.float32(_RADI * _RADI)
    mesh = plsc.VectorSubcoreMesh(core_axis_name="c", subcore_axis_name="s")

    @functools.partial(
        pl.kernel,
        mesh=mesh,
        compiler_params=pltpu.CompilerParams(needs_layout_passes=False),
        out_type=jax.ShapeDtypeStruct((B * G, D), jnp.float32),
        scratch_types=[
            pltpu.VMEM((N,), jnp.float32),
            pltpu.VMEM((N,), jnp.float32),
            pltpu.VMEM((N,), jnp.float32),
            pltpu.VMEM((QPT,), jnp.float32),
            pltpu.VMEM((QPT,), jnp.float32),
            pltpu.VMEM((QPT,), jnp.float32),
            pltpu.VMEM((QPT, _K), jnp.int32),
            pltpu.VMEM((D,), jnp.float32),
            pltpu.VMEM((D,), jnp.float32),
            pltpu.VMEM((2, _K, D), jnp.float32),
            pltpu.VMEM((QPT, D), jnp.float32),
            pltpu.SMEM((1,), jnp.int32),
            pltpu.SemaphoreType.DMA,
            pltpu.SemaphoreType.DMA,
        ],
    )
    def grouper(xyz_hbm, q_hbm, pts_hbm, al_hbm, be_hbm, out_hbm,
                x_v, y_v, z_v, qx_v, qy_v, qz_v, idx_v, al_v, be_v,
                rows_v, res_v, cnt_ref, sem0, sem1):
        wid = lax.axis_index("s") * 2 + lax.axis_index("c")
        b = wid // CPB
        qbase = (wid % CPB) * QPT
        rowbase = b * N

        pltpu.sync_copy(xyz_hbm.at[pl.ds((b * 3 + 0) * N, N)], x_v)
        pltpu.sync_copy(xyz_hbm.at[pl.ds((b * 3 + 1) * N, N)], y_v)
        pltpu.sync_copy(xyz_hbm.at[pl.ds((b * 3 + 2) * N, N)], z_v)
        pltpu.sync_copy(q_hbm.at[pl.ds((b * 3 + 0) * G + qbase, QPT)], qx_v)
        pltpu.sync_copy(q_hbm.at[pl.ds((b * 3 + 1) * G + qbase, QPT)], qy_v)
        pltpu.sync_copy(q_hbm.at[pl.ds((b * 3 + 2) * G + qbase, QPT)], qz_v)
        pltpu.sync_copy(al_hbm, al_v)
        pltpu.sync_copy(be_hbm, be_v)

        lanes = lax.iota(jnp.int32, L)

        CHUNK = 8

        def bq_block(qb, carry):
            qxv = qx_v[pl.ds(qb * L, L)]
            qyv = qy_v[pl.ds(qb * L, L)]
            qzv = qz_v[pl.ds(qb * L, L)]
            for qi in range(L):
                q = qb * L + qi
                qx = qxv[qi]
                qy = qyv[qi]
                qz = qzv[qi]
                cnt_ref[0] = jnp.int32(0)

                def chunk_body(c, carry2):
                    cnt0 = cnt_ref[0]

                    @pl.when(cnt0 < _K)
                    def _():
                        cnt = cnt0
                        for jj in range(CHUNK):
                            j = c * CHUNK + jj
                            px = x_v[pl.ds(j * L, L)]
                            py = y_v[pl.ds(j * L, L)]
                            pz = z_v[pl.ds(j * L, L)]
                            dx = qx - px
                            dy = qy - py
                            dz = qz - pz
                            d2 = (dx * dx + dy * dy) + dz * dz
                            m = d2 < RSQ
                            mi = jnp.where(m, jnp.int32(1), jnp.int32(0))
                            nh = jnp.sum(mi)
                            incl = plsc.cumsum(mi)
                            pos = cnt + incl - 1
                            wm = m & (pos < _K)
                            inds = rowbase + j * L + lanes
                            plsc.store_scatter(
                                idx_v, [jnp.full((L,), q, jnp.int32), pos],
                                inds, mask=wm,
                            )
                            cnt = cnt + nh
                        cnt_ref[0] = cnt

                    return carry2

                lax.fori_loop(0, JMAX // CHUNK, chunk_body, jnp.int32(0))
                cnt = cnt_ref[0]

                @pl.when(cnt < _K)
                def _():
                    first = jnp.full((L,), idx_v[q, pl.ds(0, L)][0], jnp.int32)
                    for half in range(_K // L):
                        sl = lanes + half * L
                        plsc.store_scatter(
                            idx_v, [jnp.full((L,), q, jnp.int32), sl], first,
                            mask=sl >= cnt,
                        )

            return carry

        lax.fori_loop(0, QPT // L, bq_block, jnp.int32(0))

        al_blk = [al_v[pl.ds(k * L, L)] for k in range(D // L)]
        be_blk = [be_v[pl.ds(k * L, L)] for k in range(D // L)]
        sems = [sem0, sem1]

        def gather(g, s):
            return pltpu.make_async_copy(
                pts_hbm.at[idx_v.at[g]], rows_v.at[s], sems[s]
            )

        gather(jnp.int32(0), 0).start()
        gather(jnp.int32(1), 1).start()

        def grp_pair(gg, carry):
            for s in range(2):
                g = gg * 2 + s
                gather(g, s).wait()
                for k in range(D // L):
                    asum = jnp.zeros((L,), jnp.float32)
                    amax = jnp.full((L,), -jnp.inf, jnp.float32)
                    for r in range(_K):
                        v = rows_v[s, r, pl.ds(k * L, L)]
                        asum = asum + v
                        amax = jnp.maximum(amax, al_blk[k] * v)
                    mean = asum * jnp.float32(1.0 / _K)
                    res_v[g, pl.ds(k * L, L)] = (
                        amax + (be_blk[k] - al_blk[k] * mean)
                    )

                @pl.when(g + 2 < QPT)
                def _():
                    gather(g + 2, s).start()
            return carry

        lax.fori_loop(0, QPT // 2, grp_pair, jnp.int32(0))

        pltpu.sync_copy(res_v, out_hbm.at[pl.ds(b * G + qbase, QPT)])

    return grouper


def kernel(xyz, points, affine_alpha, affine_beta):
    B, N, D = points.shape
    G = N // _REDUCE
    new_xyz = _fps_new_xyz(xyz, G)

    xyz_t = xyz.transpose(0, 2, 1)
    newxyz_t = new_xyz.transpose(0, 2, 1)
    feats = _make_group_sc(B, N, G, D)(
        xyz_t.reshape(-1),
        newxyz_t.reshape(-1),
        points.reshape(B * N, D),
        affine_alpha.reshape(D),
        affine_beta.reshape(D),
    )
    feats = feats.reshape(B, G, D)
    new_points = jnp.concatenate([feats.transpose(0, 2, 1), newxyz_t], axis=1)
    return (new_xyz, new_points)


# speedup vs baseline: 2.5725x; 1.4855x over previous
import functools

import jax
import jax.numpy as jnp
from jax import lax
from jax.experimental import pallas as pl
from jax.experimental.pallas import tpu as pltpu
from jax.experimental.pallas import tpu_sc as plsc

_REDUCE = 4
_K = 32
_RADI = 0.2


def _fps_body(planes_ref, out_ref):
    B = planes_ref.shape[0]
    G = out_ref.shape[1]
    sub = lax.broadcasted_iota(jnp.int32, (8, 128), 0)
    lane = lax.broadcasted_iota(jnp.int32, (8, 128), 1)
    vbase = sub * 128 + lane
    lane_row = lax.broadcasted_iota(jnp.int32, (1, 128), 1)

    def comb5(a, c):
        keep = (a[0] > c[0]) | ((a[0] == c[0]) & (a[1] < c[1]))
        return tuple(jnp.where(keep, u, w) for u, w in zip(a, c))

    def body(i, dists):
        rows = []
        for b in range(B):
            db = dists[b]
            items = [
                (
                    db[k * 8:(k + 1) * 8, :],
                    vbase + k * 1024,
                    planes_ref[b, 0, k * 8:(k + 1) * 8, :],
                    planes_ref[b, 1, k * 8:(k + 1) * 8, :],
                    planes_ref[b, 2, k * 8:(k + 1) * 8, :],
                )
                for k in range(8)
            ]
            while len(items) > 1:
                items = [
                    comb5(items[j], items[j + 1])
                    for j in range(0, len(items), 2)
                ]
            t = items[0]
            for sh in (4, 2, 1):
                t = comb5(t, tuple(pltpu.roll(a, sh, 0) for a in t))
            rows.append(tuple(a[0:1, :] for a in t))
        st = tuple(
            jnp.concatenate([rows[b][w] for b in range(B)], axis=0)
            for w in range(5)
        )
        for sh in (64, 32, 16, 8, 4, 2, 1):
            st = comb5(st, tuple(pltpu.roll(a, sh, 1) for a in st))
        xw, yw, zw = st[2], st[3], st[4]

        new_d = []
        for b in range(B):
            cxr = xw[b:b + 1, :]
            cyr = yw[b:b + 1, :]
            czr = zw[b:b + 1, :]
            row = jnp.concatenate(
                [cxr[:, 0:1], cyr[:, 0:1], czr[:, 0:1]], axis=1
            )
            out_ref[b, pl.ds(i, 1), :] = row
            dx = planes_ref[b, 0] - jnp.broadcast_to(cxr, (64, 128))
            dy = planes_ref[b, 1] - jnp.broadcast_to(cyr, (64, 128))
            dz = planes_ref[b, 2] - jnp.broadcast_to(czr, (64, 128))
            d = (dx * dx + dy * dy) + dz * dz
            new_d.append(jnp.minimum(dists[b], d))
        return tuple(new_d)

    dists0 = tuple(
        jnp.full((64, 128), 1e10, dtype=jnp.float32) for _ in range(B)
    )
    lax.fori_loop(0, G, body, dists0)


def _fps_new_xyz(xyz, G):
    B, N, _ = xyz.shape
    planes = xyz.transpose(0, 2, 1).reshape(B, 3, N // 128, 128)
    return pl.pallas_call(
        _fps_body,
        out_shape=jax.ShapeDtypeStruct((B, G, 3), jnp.float32),
    )(planes)


def _make_group_sc(B, N, G, D):
    NW = 32
    CPB = NW // B
    QPT = G // CPB
    L = 16
    JMAX = N // L
    RSQ = jnp.float32(_RADI * _RADI)
    mesh = plsc.VectorSubcoreMesh(core_axis_name="c", subcore_axis_name="s")

    @functools.partial(
        pl.kernel,
        mesh=mesh,
        compiler_params=pltpu.CompilerParams(needs_layout_passes=False),
        out_type=jax.ShapeDtypeStruct((B * G, D), jnp.float32),
        scratch_types=[
            pltpu.VMEM((N,), jnp.float32),
            pltpu.VMEM((N,), jnp.float32),
            pltpu.VMEM((N,), jnp.float32),
            pltpu.VMEM((QPT,), jnp.float32),
            pltpu.VMEM((QPT,), jnp.float32),
            pltpu.VMEM((QPT,), jnp.float32),
            pltpu.VMEM((QPT, _K), jnp.int32),
            pltpu.VMEM((D,), jnp.float32),
            pltpu.VMEM((D,), jnp.float32),
            pltpu.VMEM((2, _K, D), jnp.float32),
            pltpu.VMEM((QPT, D), jnp.float32),
            pltpu.SMEM((1,), jnp.int32),
            pltpu.SemaphoreType.DMA,
            pltpu.SemaphoreType.DMA,
        ],
    )
    def grouper(xyz_hbm, q_hbm, pts_hbm, al_hbm, be_hbm, out_hbm,
                x_v, y_v, z_v, qx_v, qy_v, qz_v, idx_v, al_v, be_v,
                rows_v, res_v, cnt_ref, sem0, sem1):
        wid = lax.axis_index("s") * 2 + lax.axis_index("c")
        b = wid // CPB
        qbase = (wid % CPB) * QPT
        rowbase = b * N

        pltpu.sync_copy(xyz_hbm.at[pl.ds((b * 3 + 0) * N, N)], x_v)
        pltpu.sync_copy(xyz_hbm.at[pl.ds((b * 3 + 1) * N, N)], y_v)
        pltpu.sync_copy(xyz_hbm.at[pl.ds((b * 3 + 2) * N, N)], z_v)
        pltpu.sync_copy(q_hbm.at[pl.ds((b * 3 + 0) * G + qbase, QPT)], qx_v)
        pltpu.sync_copy(q_hbm.at[pl.ds((b * 3 + 1) * G + qbase, QPT)], qy_v)
        pltpu.sync_copy(q_hbm.at[pl.ds((b * 3 + 2) * G + qbase, QPT)], qz_v)
        pltpu.sync_copy(al_hbm, al_v)
        pltpu.sync_copy(be_hbm, be_v)

        lanes = lax.iota(jnp.int32, L)

        CHUNK = 8

        def bq_block(qb, carry):
            qxv = qx_v[pl.ds(qb * L, L)]
            qyv = qy_v[pl.ds(qb * L, L)]
            qzv = qz_v[pl.ds(qb * L, L)]
            for qi in range(L):
                q = qb * L + qi
                qx = qxv[qi]
                qy = qyv[qi]
                qz = qzv[qi]
                cnt_ref[0] = jnp.int32(0)

                def chunk_body(c, carry2):
                    cnt0 = cnt_ref[0]

                    @pl.when(cnt0 < _K)
                    def _():
                        cnt = cnt0
                        for jj in range(CHUNK):
                            j = c * CHUNK + jj
                            px = x_v[pl.ds(j * L, L)]
                            py = y_v[pl.ds(j * L, L)]
                            pz = z_v[pl.ds(j * L, L)]
                            dx = qx - px
                            dy = qy - py
                            dz = qz - pz
                            d2 = (dx * dx + dy * dy) + dz * dz
                            m = d2 < RSQ
                            mi = jnp.where(m, jnp.int32(1), jnp.int32(0))
                            nh = jnp.sum(mi)
                            incl = plsc.cumsum(mi)
                            pos = cnt + incl - 1
                            wm = m & (pos < _K)
                            inds = rowbase + j * L + lanes
                            plsc.store_scatter(
                                idx_v, [jnp.full((L,), q, jnp.int32), pos],
                                inds, mask=wm,
                            )
                            cnt = cnt + nh
                        cnt_ref[0] = cnt

                    return carry2

                lax.fori_loop(0, JMAX // CHUNK, chunk_body, jnp.int32(0))
                cnt = cnt_ref[0]

                @pl.when(cnt < _K)
                def _():
                    first = jnp.full((L,), idx_v[q, pl.ds(0, L)][0], jnp.int32)
                    for half in range(_K // L):
                        sl = lanes + half * L
                        plsc.store_scatter(
                            idx_v, [jnp.full((L,), q, jnp.int32), sl], first,
                            mask=sl >= cnt,
                        )

            return carry

        lax.fori_loop(0, QPT // L, bq_block, jnp.int32(0))

        al_blk = [al_v[pl.ds(k * L, L)] for k in range(D // L)]
        be_blk = [be_v[pl.ds(k * L, L)] for k in range(D // L)]
        sems = [sem0, sem1]

        def gather(g, s):
            return pltpu.make_async_copy(
                pts_hbm.at[idx_v.at[g]], rows_v.at[s], sems[s]
            )

        gather(jnp.int32(0), 0).start()
        gather(jnp.int32(1), 1).start()

        def grp_pair(gg, carry):
            for s in range(2):
                g = gg * 2 + s
                gather(g, s).wait()
                for k in range(D // L):
                    asum = jnp.zeros((L,), jnp.float32)
                    amax = jnp.full((L,), -jnp.inf, jnp.float32)
                    for r in range(_K):
                        v = rows_v[s, r, pl.ds(k * L, L)]
                        asum = asum + v
                        amax = jnp.maximum(amax, al_blk[k] * v)
                    mean = asum * jnp.float32(1.0 / _K)
                    res_v[g, pl.ds(k * L, L)] = (
                        amax + (be_blk[k] - al_blk[k] * mean)
                    )

                @pl.when(g + 2 < QPT)
                def _():
                    gather(g + 2, s).start()
            return carry

        lax.fori_loop(0, QPT // 2, grp_pair, jnp.int32(0))

        pltpu.sync_copy(res_v, out_hbm.at[pl.ds(b * G + qbase, QPT)])

    return grouper


def kernel(xyz, points, affine_alpha, affine_beta):
    B, N, D = points.shape
    G = N // _REDUCE
    new_xyz = _fps_new_xyz(xyz, G)

    xyz_t = xyz.transpose(0, 2, 1)
    newxyz_t = new_xyz.transpose(0, 2, 1)
    feats = _make_group_sc(B, N, G, D)(
        xyz_t.reshape(-1),
        newxyz_t.reshape(-1),
        points.reshape(B * N, D),
        affine_alpha.reshape(D),
        affine_beta.reshape(D),
    )
    feats = feats.reshape(B, G, D)
    new_points = jnp.concatenate([feats.transpose(0, 2, 1), newxyz_t], axis=1)
    return (new_xyz, new_points)
